# Initial kernel scaffold; baseline (speedup 1.0000x reference)
import jax
import jax.numpy as jnp
from jax.experimental import pallas as pl


def kernel(x_nchw, w1, b1, w2, b2, wf1, bf1, wf2, bf2, wf3, bf3):
    raise NotImplementedError("write your pallas kernel here")


# speedup vs baseline: 5.6071x; 5.6071x over previous
import jax
import jax.numpy as jnp
from jax.experimental import pallas as pl
from jax.experimental.pallas import tpu as pltpu

HW_IN = 32
XCOLS = 96
NPAD = 128
NFUSE = 2 * NPAD
P1 = 14
P2 = 5
NCLASS = 102
KS = 5


def _lenet_kernel(x_ref,
                  w1_ref, b1_ref, w2_ref, b2_ref,
                  wf1_ref, bf1_ref, wf2_ref, bf2_ref, wf3_ref, bf3_ref,
                  o_ref, h1_ref):
    f32 = jnp.float32
    bf16 = jnp.bfloat16
    bt = o_ref.shape[0]

    acc_e = jnp.zeros((P1, bt, NFUSE), f32)
    acc_o = jnp.zeros((P1, bt, NFUSE), f32)
    for kh in range(KS):
        m = kh // 2
        if kh % 2 == 0:
            xs = x_ref[:, pl.ds(m, P1)]
            res = jnp.dot(xs.reshape(2 * P1 * bt, XCOLS), w1_ref[kh],
                          preferred_element_type=f32)
            res = res.reshape(2, P1, bt, NFUSE)
            acc_e = acc_e + res[0]
            acc_o = acc_o + res[1]
        else:
            xs = x_ref[:, pl.ds(m, P1 + 1)]
            res = jnp.dot(xs.reshape(2 * (P1 + 1) * bt, XCOLS), w1_ref[kh],
                          preferred_element_type=f32)
            res = res.reshape(2, P1 + 1, bt, NFUSE)
            acc_e = acc_e + res[1, 0:P1]
            acc_o = acc_o + res[0, 1:P1 + 1]
    m1 = jnp.maximum(jnp.maximum(acc_e[..., :NPAD], acc_e[..., NPAD:]),
                     jnp.maximum(acc_o[..., :NPAD], acc_o[..., NPAD:]))
    h1_ref[...] = jnp.maximum(m1 + b1_ref[...], 0.0).astype(bf16)

    acc2_e = jnp.zeros((P2, bt, NFUSE), f32)
    acc2_o = jnp.zeros((P2, bt, NFUSE), f32)
    for kh in range(KS):
        hs = h1_ref[pl.ds(kh, 2 * P2)]
        res = jnp.dot(hs.reshape(2 * P2 * bt, NPAD), w2_ref[kh],
                      preferred_element_type=f32)
        res = res.reshape(P2, 2, bt, NFUSE)
        acc2_e = acc2_e + res[:, 0]
        acc2_o = acc2_o + res[:, 1]
    m2 = jnp.maximum(jnp.maximum(acc2_e[..., :NPAD], acc2_e[..., NPAD:]),
                     jnp.maximum(acc2_o[..., :NPAD], acc2_o[..., NPAD:]))
    h2 = jnp.maximum(m2 + b2_ref[...], 0.0).astype(bf16)

    a = jnp.zeros((bt, NPAD), f32)
    for r in range(P2):
        a = a + jnp.dot(h2[r], wf1_ref[r], preferred_element_type=f32)
    a = jnp.maximum(a + bf1_ref[...], 0.0).astype(bf16)
    a = jnp.maximum(jnp.dot(a, wf2_ref[...], preferred_element_type=f32)
                    + bf2_ref[...], 0.0).astype(bf16)
    o_ref[...] = (jnp.dot(a, wf3_ref[...], preferred_element_type=f32)
                  + bf3_ref[...])


def kernel(x_nchw, w1, b1, w2, b2, wf1, bf1, wf2, bf2, wf3, bf3):
    B = x_nchw.shape[0]
    bf16 = jnp.bfloat16
    Bt = B if B <= 128 else 128
    nblk = -(-B // Bt)
    Bp = nblk * Bt

    x = jnp.transpose(x_nchw.astype(bf16), (0, 2, 3, 1)).reshape(B, HW_IN, XCOLS)
    x = x.reshape(B, HW_IN // 2, 2, XCOLS)
    x = jnp.transpose(x, (2, 1, 0, 3))
    if Bp != B:
        x = jnp.pad(x, ((0, 0), (0, 0), (0, Bp - B), (0, 0)))

    weights = (w1.astype(bf16), b1, w2.astype(bf16), b2,
               wf1.astype(bf16), bf1, wf2.astype(bf16), bf2,
               wf3.astype(bf16), bf3)

    def _const_spec(a):
        return pl.BlockSpec(a.shape, lambda b: (0,) * a.ndim)

    in_specs = [pl.BlockSpec((2, HW_IN // 2, Bt, XCOLS),
                             lambda b: (0, 0, b, 0))]
    in_specs += [_const_spec(a) for a in weights]

    out = pl.pallas_call(
        _lenet_kernel,
        out_shape=jax.ShapeDtypeStruct((Bp, NPAD), jnp.float32),
        grid=(nblk,),
        in_specs=in_specs,
        out_specs=pl.BlockSpec((Bt, NPAD), lambda b: (b, 0)),
        scratch_shapes=[pltpu.VMEM((P1, Bt, NPAD), bf16)],
        compiler_params=pltpu.CompilerParams(
            dimension_semantics=("parallel",)),
    )(x, *weights)
    return out[:B, :NCLASS]


# speedup vs baseline: 7.7469x; 1.3816x over previous
import jax
import jax.numpy as jnp
from jax.experimental import pallas as pl
from jax.experimental.pallas import tpu as pltpu

HW_IN = 32
XCOLS = 96
NPAD = 128
NFUSE = 2 * NPAD
P1 = 14
P2 = 5
NCLASS = 102
KS = 5
KCAT = KS * NPAD


def _lenet_kernel(x_ref,
                  w1_ref, b1_ref, w2_ref, b2_ref,
                  wf1_ref, bf1_ref, wf2_ref, bf2_ref, wf3_ref, bf3_ref,
                  o_ref):
    f32 = jnp.float32
    bf16 = jnp.bfloat16
    bt = o_ref.shape[0]

    xcat = jnp.concatenate([x_ref[pl.ds(k, 2 * P1)] for k in range(KS)],
                           axis=-1)
    res = jnp.dot(xcat.reshape(2 * P1 * bt, KCAT), w1_ref[...],
                  preferred_element_type=f32)
    res = res.reshape(P1, 2, bt, NFUSE)
    m1 = jnp.maximum(jnp.maximum(res[:, 0, :, :NPAD], res[:, 0, :, NPAD:]),
                     jnp.maximum(res[:, 1, :, :NPAD], res[:, 1, :, NPAD:]))
    h1 = jnp.maximum(m1 + b1_ref[...], 0.0).astype(bf16)

    hcat = jnp.concatenate([h1[k:k + 2 * P2] for k in range(KS)],
                           axis=-1)
    res2 = jnp.dot(hcat.reshape(2 * P2 * bt, KCAT), w2_ref[...],
                   preferred_element_type=f32)
    res2 = res2.reshape(P2, 2, bt, NFUSE)
    m2 = jnp.maximum(jnp.maximum(res2[:, 0, :, :NPAD], res2[:, 0, :, NPAD:]),
                     jnp.maximum(res2[:, 1, :, :NPAD], res2[:, 1, :, NPAD:]))
    h2 = jnp.maximum(m2 + b2_ref[...], 0.0).astype(bf16)

    hf = jnp.concatenate([h2[r] for r in range(P2)], axis=-1)
    a = jnp.dot(hf, wf1_ref[...], preferred_element_type=f32)
    a = jnp.maximum(a + bf1_ref[...], 0.0).astype(bf16)
    a = jnp.maximum(jnp.dot(a, wf2_ref[...], preferred_element_type=f32)
                    + bf2_ref[...], 0.0).astype(bf16)
    o_ref[...] = (jnp.dot(a, wf3_ref[...], preferred_element_type=f32)
                  + bf3_ref[...])


def kernel(x_nchw, w1, b1, w2, b2, wf1, bf1, wf2, bf2, wf3, bf3):
    B = x_nchw.shape[0]
    bf16 = jnp.bfloat16
    Bt = B if B <= 128 else 128
    nblk = -(-B // Bt)
    Bp = nblk * Bt

    x = jnp.transpose(x_nchw.astype(bf16), (0, 2, 3, 1)).reshape(B, HW_IN, XCOLS)
    x = jnp.transpose(x, (1, 0, 2))
    x = jnp.pad(x, ((0, 0), (0, Bp - B), (0, NPAD - XCOLS)))

    w1c = jnp.pad(w1, ((0, 0), (0, NPAD - XCOLS), (0, 0)))
    w1c = w1c.reshape(KCAT, NFUSE).astype(bf16)
    w2c = w2.reshape(KCAT, NFUSE).astype(bf16)
    wf1c = wf1.reshape(KCAT, NPAD).astype(bf16)
    weights = (w1c, b1, w2c, b2, wf1c, bf1,
               wf2.astype(bf16), bf2, wf3.astype(bf16), bf3)

    def _const_spec(a):
        return pl.BlockSpec(a.shape, lambda b: (0,) * a.ndim)

    in_specs = [pl.BlockSpec((HW_IN, Bt, NPAD), lambda b: (0, b, 0))]
    in_specs += [_const_spec(a) for a in weights]

    out = pl.pallas_call(
        _lenet_kernel,
        out_shape=jax.ShapeDtypeStruct((Bp, NPAD), jnp.float32),
        grid=(nblk,),
        in_specs=in_specs,
        out_specs=pl.BlockSpec((Bt, NPAD), lambda b: (b, 0)),
        compiler_params=pltpu.CompilerParams(
            dimension_semantics=("parallel",)),
    )(x, *weights)
    return out[:B, :NCLASS]


# speedup vs baseline: 8.1006x; 1.0457x over previous
import jax
import jax.numpy as jnp
from jax.experimental import pallas as pl
from jax.experimental.pallas import tpu as pltpu

HW_IN = 32
XCOLS = 96
NPAD = 128
NFUSE = 2 * NPAD
P1 = 14
P2 = 5
NCLASS = 102
KS = 5
KCAT = KS * NPAD


def _lenet_kernel(x_ref,
                  w1_ref, b1_ref, w2_ref, b2_ref,
                  wf1_ref, bf1_ref, wf2_ref, bf2_ref, wf3_ref, bf3_ref,
                  o_ref):
    f32 = jnp.float32
    bf16 = jnp.bfloat16
    bt = o_ref.shape[0]

    xcat = jnp.concatenate([x_ref[pl.ds(k, 2 * P1)] for k in range(KS)],
                           axis=-1)
    res = jnp.dot(xcat.reshape(2 * P1 * bt, KCAT), w1_ref[...],
                  preferred_element_type=f32)
    res = res.reshape(P1, 2, bt, NFUSE)
    m1 = jnp.maximum(jnp.maximum(res[:, 0, :, :NPAD], res[:, 0, :, NPAD:]),
                     jnp.maximum(res[:, 1, :, :NPAD], res[:, 1, :, NPAD:]))
    h1 = jnp.maximum(m1 + b1_ref[...], 0.0).astype(bf16)

    hcat = jnp.concatenate([h1[k:k + 2 * P2] for k in range(KS)],
                           axis=-1)
    res2 = jnp.dot(hcat.reshape(2 * P2 * bt, KCAT), w2_ref[...],
                   preferred_element_type=f32)
    res2 = res2.reshape(P2, 2, bt, NFUSE)
    m2 = jnp.maximum(jnp.maximum(res2[:, 0, :, :NPAD], res2[:, 0, :, NPAD:]),
                     jnp.maximum(res2[:, 1, :, :NPAD], res2[:, 1, :, NPAD:]))
    h2 = jnp.maximum(m2 + b2_ref[...], 0.0).astype(bf16)

    hf = jnp.concatenate([h2[r] for r in range(P2)], axis=-1)
    a = jnp.dot(hf, wf1_ref[...], preferred_element_type=f32)
    a = jnp.maximum(a + bf1_ref[...], 0.0).astype(bf16)
    a = jnp.maximum(jnp.dot(a, wf2_ref[...], preferred_element_type=f32)
                    + bf2_ref[...], 0.0).astype(bf16)
    o_ref[...] = (jnp.dot(a, wf3_ref[...], preferred_element_type=f32)
                  + bf3_ref[...])


def kernel(x_nchw, w1, b1, w2, b2, wf1, bf1, wf2, bf2, wf3, bf3):
    B = x_nchw.shape[0]
    bf16 = jnp.bfloat16
    Bt = B if B <= 256 else 256
    nblk = -(-B // Bt)
    Bp = nblk * Bt

    x = jnp.transpose(x_nchw.astype(bf16), (0, 2, 3, 1)).reshape(B, HW_IN, XCOLS)
    x = jnp.transpose(x, (1, 0, 2))
    x = jnp.pad(x, ((0, 0), (0, Bp - B), (0, NPAD - XCOLS)))

    w1c = jnp.pad(w1, ((0, 0), (0, NPAD - XCOLS), (0, 0)))
    w1c = w1c.reshape(KCAT, NFUSE).astype(bf16)
    w2c = w2.reshape(KCAT, NFUSE).astype(bf16)
    wf1c = wf1.reshape(KCAT, NPAD).astype(bf16)
    weights = (w1c, b1, w2c, b2, wf1c, bf1,
               wf2.astype(bf16), bf2, wf3.astype(bf16), bf3)

    def _const_spec(a):
        return pl.BlockSpec(a.shape, lambda b: (0,) * a.ndim)

    in_specs = [pl.BlockSpec((HW_IN, Bt, NPAD), lambda b: (0, b, 0))]
    in_specs += [_const_spec(a) for a in weights]

    out = pl.pallas_call(
        _lenet_kernel,
        out_shape=jax.ShapeDtypeStruct((Bp, NPAD), jnp.float32),
        grid=(nblk,),
        in_specs=in_specs,
        out_specs=pl.BlockSpec((Bt, NPAD), lambda b: (b, 0)),
        compiler_params=pltpu.CompilerParams(
            dimension_semantics=("parallel",)),
    )(x, *weights)
    return out[:B, :NCLASS]


# speedup vs baseline: 9.2460x; 1.1414x over previous
import jax
import jax.numpy as jnp
from jax.experimental import pallas as pl
from jax.experimental.pallas import tpu as pltpu

HW_IN = 32
XCOLS = 96
NPAD = 128
NFUSE = 2 * NPAD
P1 = 14
P2 = 5
NCLASS = 102
KS = 5
IC1 = 3
K1 = IC1 * KS * HW_IN
KCAT = KS * NPAD


def _lenet_kernel(x_ref,
                  w1_ref, b1_ref, w2_ref, b2_ref,
                  wf1_ref, bf1_ref, wf2_ref, bf2_ref, wf3_ref, bf3_ref,
                  o_ref):
    f32 = jnp.float32
    bf16 = jnp.bfloat16
    bt = o_ref.shape[0]

    xt = jnp.transpose(x_ref[...], (1, 0, 2)).astype(bf16)

    xcat = jnp.concatenate(
        [xt[c * HW_IN + kh:c * HW_IN + kh + 2 * P1]
         for c in range(IC1) for kh in range(KS)], axis=-1)
    res = jnp.dot(xcat.reshape(2 * P1 * bt, K1), w1_ref[...],
                  preferred_element_type=f32)
    res = res.reshape(P1, 2, bt, NFUSE)
    m1 = jnp.maximum(jnp.maximum(res[:, 0, :, :NPAD], res[:, 0, :, NPAD:]),
                     jnp.maximum(res[:, 1, :, :NPAD], res[:, 1, :, NPAD:]))
    h1 = jnp.maximum(m1 + b1_ref[...], 0.0).astype(bf16)

    hcat = jnp.concatenate([h1[k:k + 2 * P2] for k in range(KS)],
                           axis=-1)
    res2 = jnp.dot(hcat.reshape(2 * P2 * bt, KCAT), w2_ref[...],
                   preferred_element_type=f32)
    res2 = res2.reshape(P2, 2, bt, NFUSE)
    m2 = jnp.maximum(jnp.maximum(res2[:, 0, :, :NPAD], res2[:, 0, :, NPAD:]),
                     jnp.maximum(res2[:, 1, :, :NPAD], res2[:, 1, :, NPAD:]))
    h2 = jnp.maximum(m2 + b2_ref[...], 0.0).astype(bf16)

    hf = jnp.concatenate([h2[r] for r in range(P2)], axis=-1)
    a = jnp.dot(hf, wf1_ref[...], preferred_element_type=f32)
    a = jnp.maximum(a + bf1_ref[...], 0.0).astype(bf16)
    a = jnp.maximum(jnp.dot(a, wf2_ref[...], preferred_element_type=f32)
                    + bf2_ref[...], 0.0).astype(bf16)
    o_ref[...] = (jnp.dot(a, wf3_ref[...], preferred_element_type=f32)
                  + bf3_ref[...])


def kernel(x_nchw, w1, b1, w2, b2, wf1, bf1, wf2, bf2, wf3, bf3):
    B = x_nchw.shape[0]
    bf16 = jnp.bfloat16
    Bt = B if B <= 256 else 256
    nblk = -(-B // Bt)
    Bp = nblk * Bt

    x = x_nchw.reshape(B, IC1 * HW_IN, HW_IN)
    if Bp != B:
        x = jnp.pad(x, ((0, Bp - B), (0, 0), (0, 0)))

    w1c = (w1.reshape(KS, HW_IN, IC1, NFUSE).transpose(2, 0, 1, 3)
           .reshape(K1, NFUSE).astype(bf16))
    w2c = w2.reshape(KCAT, NFUSE).astype(bf16)
    wf1c = wf1.reshape(KCAT, NPAD).astype(bf16)
    weights = (w1c, b1, w2c, b2, wf1c, bf1,
               wf2.astype(bf16), bf2, wf3.astype(bf16), bf3)

    def _const_spec(a):
        return pl.BlockSpec(a.shape, lambda b: (0,) * a.ndim)

    in_specs = [pl.BlockSpec((Bt, IC1 * HW_IN, HW_IN), lambda b: (b, 0, 0))]
    in_specs += [_const_spec(a) for a in weights]

    out = pl.pallas_call(
        _lenet_kernel,
        out_shape=jax.ShapeDtypeStruct((Bp, NPAD), jnp.float32),
        grid=(nblk,),
        in_specs=in_specs,
        out_specs=pl.BlockSpec((Bt, NPAD), lambda b: (b, 0)),
        compiler_params=pltpu.CompilerParams(
            dimension_semantics=("parallel",)),
    )(x, *weights)
    return out[:B, :NCLASS]


# speedup vs baseline: 12.5442x; 1.3567x over previous
import jax
import jax.numpy as jnp
from jax.experimental import pallas as pl
from jax.experimental.pallas import tpu as pltpu

HW_IN = 32
NPAD = 128
NFUSE = 2 * NPAD
P1 = 14
P2 = 5
NCLASS = 102
KS = 5
IC1 = 3
NQ = 24
K1 = 2 * IC1 * NPAD
N1 = 4 * NFUSE
KCAT = KS * NPAD


def _lenet_kernel(x_ref,
                  w1_ref, b1_ref, w2_ref, b2_ref,
                  wf1_ref, bf1_ref, wf2_ref, bf2_ref, wf3_ref, bf3_ref,
                  o_ref):
    f32 = jnp.float32
    bf16 = jnp.bfloat16
    bt = o_ref.shape[0]

    xq = jnp.transpose(x_ref[...], (1, 0, 2)).astype(bf16)

    xcat = jnp.concatenate(
        [xq[8 * c + qr:8 * c + qr + 7] for c in range(IC1) for qr in (0, 1)],
        axis=-1)
    res = jnp.dot(xcat.reshape(7 * bt, K1), w1_ref[...],
                  preferred_element_type=f32)
    res = res.reshape(7, bt, N1)
    ev = jnp.maximum(
        jnp.maximum(res[..., 0 * NPAD:1 * NPAD], res[..., 1 * NPAD:2 * NPAD]),
        jnp.maximum(res[..., 2 * NPAD:3 * NPAD], res[..., 3 * NPAD:4 * NPAD]))
    od = jnp.maximum(
        jnp.maximum(res[..., 4 * NPAD:5 * NPAD], res[..., 5 * NPAD:6 * NPAD]),
        jnp.maximum(res[..., 6 * NPAD:7 * NPAD], res[..., 7 * NPAD:8 * NPAD]))
    m1 = jnp.stack([ev, od], axis=1).reshape(P1, bt, NPAD)
    h1 = jnp.maximum(m1 + b1_ref[...], 0.0).astype(bf16)

    hcat = jnp.concatenate([h1[k:k + 2 * P2] for k in range(KS)],
                           axis=-1)
    res2 = jnp.dot(hcat.reshape(2 * P2 * bt, KCAT), w2_ref[...],
                   preferred_element_type=f32)
    res2 = res2.reshape(P2, 2, bt, NFUSE)
    m2 = jnp.maximum(jnp.maximum(res2[:, 0, :, :NPAD], res2[:, 0, :, NPAD:]),
                     jnp.maximum(res2[:, 1, :, :NPAD], res2[:, 1, :, NPAD:]))
    h2 = jnp.maximum(m2 + b2_ref[...], 0.0).astype(bf16)

    hf = jnp.concatenate([h2[r] for r in range(P2)], axis=-1)
    a = jnp.dot(hf, wf1_ref[...], preferred_element_type=f32)
    a = jnp.maximum(a + bf1_ref[...], 0.0).astype(bf16)
    a = jnp.maximum(jnp.dot(a, wf2_ref[...], preferred_element_type=f32)
                    + bf2_ref[...], 0.0).astype(bf16)
    o_ref[...] = (jnp.dot(a, wf3_ref[...], preferred_element_type=f32)
                  + bf3_ref[...])


def _band_conv1(w1):
    q = jnp.arange(2)[:, None, None]
    h4 = jnp.arange(4)[None, :, None]
    m = jnp.arange(4)[None, None, :]
    kh = 4 * q + h4 - m
    valid = (kh >= 0) & (kh < KS)
    khc = jnp.clip(kh, 0, KS - 1).reshape(-1)
    lane = (jnp.arange(HW_IN)[None, :] * IC1
            + jnp.arange(IC1)[:, None]).reshape(-1)
    a = w1[khc][:, lane, :]
    a = a.reshape(2, 4, 4, IC1, HW_IN, NFUSE)
    a = jnp.where(valid[..., None, None, None], a, 0.0)
    a = a.transpose(3, 0, 1, 4, 2, 5)
    return a.reshape(K1, N1)


def kernel(x_nchw, w1, b1, w2, b2, wf1, bf1, wf2, bf2, wf3, bf3):
    B = x_nchw.shape[0]
    bf16 = jnp.bfloat16
    Bt = B if B <= 256 else 256
    nblk = -(-B // Bt)
    Bp = nblk * Bt

    x = x_nchw.reshape(B, NQ, NPAD)
    if Bp != B:
        x = jnp.pad(x, ((0, Bp - B), (0, 0), (0, 0)))

    w1c = _band_conv1(w1).astype(bf16)
    w2c = w2.reshape(KCAT, NFUSE).astype(bf16)
    wf1c = wf1.reshape(KCAT, NPAD).astype(bf16)
    weights = (w1c, b1, w2c, b2, wf1c, bf1,
               wf2.astype(bf16), bf2, wf3.astype(bf16), bf3)

    def _const_spec(a):
        return pl.BlockSpec(a.shape, lambda b: (0,) * a.ndim)

    in_specs = [pl.BlockSpec((Bt, NQ, NPAD), lambda b: (b, 0, 0))]
    in_specs += [_const_spec(a) for a in weights]

    out = pl.pallas_call(
        _lenet_kernel,
        out_shape=jax.ShapeDtypeStruct((Bp, NPAD), jnp.float32),
        grid=(nblk,),
        in_specs=in_specs,
        out_specs=pl.BlockSpec((Bt, NPAD), lambda b: (b, 0)),
        compiler_params=pltpu.CompilerParams(
            dimension_semantics=("parallel",)),
    )(x, *weights)
    return out[:B, :NCLASS]
